# baseline (device time: 30664 ns/iter reference)
import jax
import jax.numpy as jnp
from jax import lax
from jax.experimental import pallas as pl
from jax.experimental.pallas import tpu as pltpu

N_DEV = 8
B, SQ, D_MODEL = 2, 128, 512
H_PER, DH = 4, 64
ROWS = B * SQ
ROWS_PER = ROWS // N_DEV


def kernel(x, Wq, K_ext, V_ext, Wo):
    my = lax.axis_index("i")
    Ks = lax.dynamic_slice_in_dim(K_ext, my * H_PER, H_PER, axis=2)
    Vs = lax.dynamic_slice_in_dim(V_ext, my * H_PER, H_PER, axis=2)
    Ks = jnp.transpose(Ks, (0, 2, 1, 3))
    Vs = jnp.transpose(Vs, (0, 2, 1, 3))
    xf = x.reshape(ROWS, D_MODEL)

    def body(x_ref, wq_ref, k_ref, v_ref, wo_ref, out_ref,
             partial_ref, comm_a, comm_b, local_sems,
             send_a, recv_a, send_b, recv_b):
        my_pos = lax.axis_index("i")

        def slice_rdma_a(s):
            return pltpu.make_async_remote_copy(
                src_ref=partial_ref.at[s],
                dst_ref=comm_a.at[my_pos],
                send_sem=send_a.at[s],
                recv_sem=recv_a.at[my_pos],
                device_id=(s,),
                device_id_type=pl.DeviceIdType.MESH,
            )

        q_all = jnp.dot(x_ref[:, :], wq_ref[:, :],
                        preferred_element_type=jnp.float32)
        for s in range(N_DEV):
            b, r0 = s * ROWS_PER // SQ, (s * ROWS_PER) % SQ
            ctx_parts = []
            for h in range(H_PER):
                q = q_all[s * ROWS_PER:(s + 1) * ROWS_PER,
                          h * DH:(h + 1) * DH]
                k = k_ref[b, h]
                v = v_ref[b, h]
                sc = lax.dot_general(
                    q, k, (((1,), (1,)), ((), ())),
                    preferred_element_type=jnp.float32) * 0.125
                m = jnp.max(sc, axis=-1, keepdims=True)
                e = jnp.exp(sc - m)
                w = e / jnp.sum(e, axis=-1, keepdims=True)
                ctx_parts.append(jnp.dot(w, v,
                                         preferred_element_type=jnp.float32))
            ctx_s = jnp.concatenate(ctx_parts, axis=1)
            p_s = jnp.dot(ctx_s, wo_ref[:, :],
                          preferred_element_type=jnp.float32)
            partial_ref[s] = p_s

            @pl.when(s != my_pos)
            def _():
                slice_rdma_a(s).start()

            @pl.when(s == my_pos)
            def _():
                comm_a[s] = p_s

        for p in range(N_DEV):
            @pl.when(p != my_pos)
            def _():
                pltpu.make_async_remote_copy(
                    src_ref=partial_ref.at[p],
                    dst_ref=comm_a.at[p],
                    send_sem=send_a.at[p],
                    recv_sem=recv_a.at[p],
                    device_id=(p,),
                    device_id_type=pl.DeviceIdType.MESH,
                ).wait_recv()
        red = comm_a[0]
        for p in range(1, N_DEV):
            red = red + comm_a[p]
        comm_b[:, :] = red

        own_b = pltpu.make_async_copy(
            comm_b, out_ref.at[pl.ds(my_pos * ROWS_PER, ROWS_PER), :],
            local_sems.at[0])
        own_b.start()

        for t in range(N_DEV):
            @pl.when(t != my_pos)
            def _():
                pltpu.make_async_remote_copy(
                    src_ref=comm_b,
                    dst_ref=out_ref.at[pl.ds(my_pos * ROWS_PER, ROWS_PER), :],
                    send_sem=send_b.at[t],
                    recv_sem=recv_b.at[my_pos],
                    device_id=(t,),
                    device_id_type=pl.DeviceIdType.MESH,
                ).start()
        own_b.wait()
        for p in range(N_DEV):
            @pl.when(p != my_pos)
            def _():
                pltpu.make_async_remote_copy(
                    src_ref=comm_b,
                    dst_ref=out_ref.at[p * ROWS_PER:(p + 1) * ROWS_PER, :],
                    send_sem=send_b.at[p],
                    recv_sem=recv_b.at[p],
                    device_id=(p,),
                    device_id_type=pl.DeviceIdType.MESH,
                ).wait_recv()
        for s in range(N_DEV):
            @pl.when(s != my_pos)
            def _():
                slice_rdma_a(s).wait_send()
                pltpu.make_async_remote_copy(
                    src_ref=comm_b,
                    dst_ref=out_ref.at[pl.ds(my_pos * ROWS_PER, ROWS_PER), :],
                    send_sem=send_b.at[s],
                    recv_sem=recv_b.at[my_pos],
                    device_id=(s,),
                    device_id_type=pl.DeviceIdType.MESH,
                ).wait_send()

    out = pl.pallas_call(
        body,
        out_shape=jax.ShapeDtypeStruct((ROWS, D_MODEL), jnp.float32),
        in_specs=[pl.BlockSpec(memory_space=pltpu.VMEM)] * 5,
        out_specs=pl.BlockSpec(memory_space=pltpu.VMEM),
        scratch_shapes=[
            pltpu.VMEM((N_DEV, ROWS_PER, D_MODEL), jnp.float32),
            pltpu.VMEM((N_DEV, ROWS_PER, D_MODEL), jnp.float32),
            pltpu.VMEM((ROWS_PER, D_MODEL), jnp.float32),
            pltpu.SemaphoreType.DMA((1,)),
            pltpu.SemaphoreType.DMA((N_DEV,)),
            pltpu.SemaphoreType.DMA((N_DEV,)),
            pltpu.SemaphoreType.DMA((N_DEV,)),
            pltpu.SemaphoreType.DMA((N_DEV,)),
        ],
    )(xf, Wq, Ks, Vs, Wo)
    return out.reshape(B, SQ, D_MODEL)


# device time: 20111 ns/iter; 1.5247x vs baseline; 1.5247x over previous
import jax
import jax.numpy as jnp
from jax import lax
from jax.experimental import pallas as pl
from jax.experimental.pallas import tpu as pltpu

N_DEV = 8
B, SQ, D_MODEL = 2, 128, 512
H_PER, DH = 4, 64
ROWS = B * SQ
ROWS_PER = ROWS // N_DEV


def kernel(x, Wq, K_ext, V_ext, Wo):
    my = lax.axis_index("i")
    Ks = lax.dynamic_slice_in_dim(K_ext, my * H_PER, H_PER, axis=2)
    Vs = lax.dynamic_slice_in_dim(V_ext, my * H_PER, H_PER, axis=2)
    Ks = jnp.transpose(Ks, (0, 2, 1, 3))
    Vs = jnp.transpose(Vs, (0, 2, 1, 3))
    xf = x.reshape(ROWS, D_MODEL)

    def body(x_ref, wq_ref, k_ref, v_ref, wo_ref, out_ref,
             partial_ref, comm_a, comm_b, gather_ref, local_sems,
             send_a, recv_a, send_b, recv_b):
        my_pos = lax.axis_index("i")

        q_all = jnp.dot(x_ref[:, :], wq_ref[:, :],
                        preferred_element_type=jnp.float32)
        for b in range(B):
            ctx_parts = []
            for h in range(H_PER):
                q = q_all[b * SQ:(b + 1) * SQ, h * DH:(h + 1) * DH]
                k = k_ref[b, h]
                v = v_ref[b, h]
                s = lax.dot_general(
                    q, k, (((1,), (1,)), ((), ())),
                    preferred_element_type=jnp.float32) * 0.125
                m = jnp.max(s, axis=-1, keepdims=True)
                e = jnp.exp(s - m)
                w = e / jnp.sum(e, axis=-1, keepdims=True)
                ctx_parts.append(jnp.dot(w, v,
                                         preferred_element_type=jnp.float32))
            ctx_b = jnp.concatenate(ctx_parts, axis=1)
            p_b = jnp.dot(ctx_b, wo_ref[:, :],
                          preferred_element_type=jnp.float32)
            partial_ref[4 * b:4 * (b + 1)] = p_b.astype(jnp.bfloat16).reshape(
                4, ROWS_PER, D_MODEL)

        own_a = pltpu.make_async_copy(
            partial_ref.at[my_pos], comm_a.at[0], local_sems.at[0])
        own_a.start()
        sends_a = []
        for o in range(1, N_DEV):
            t = lax.rem(my_pos + o, N_DEV)
            rdma = pltpu.make_async_remote_copy(
                src_ref=partial_ref.at[t],
                dst_ref=comm_a.at[o],
                send_sem=send_a.at[o],
                recv_sem=recv_a.at[o],
                device_id=(t,),
                device_id_type=pl.DeviceIdType.MESH,
            )
            rdma.start()
            sends_a.append(rdma)
        own_a.wait()
        for o in range(1, N_DEV):
            pltpu.make_async_remote_copy(
                src_ref=comm_a.at[o],
                dst_ref=comm_a.at[o],
                send_sem=send_a.at[o],
                recv_sem=recv_a.at[o],
                device_id=(my_pos,),
                device_id_type=pl.DeviceIdType.MESH,
            ).wait_recv()
        red = comm_a[0].astype(jnp.float32)
        for o in range(1, N_DEV):
            red = red + comm_a[o].astype(jnp.float32)
        comm_b[:, :] = red.astype(jnp.bfloat16)

        sends_b = []
        for o in range(1, N_DEV):
            t = lax.rem(my_pos + o, N_DEV)
            rdma = pltpu.make_async_remote_copy(
                src_ref=comm_b,
                dst_ref=gather_ref.at[my_pos],
                send_sem=send_b.at[o],
                recv_sem=recv_b.at[my_pos],
                device_id=(t,),
                device_id_type=pl.DeviceIdType.MESH,
            )
            rdma.start()
            sends_b.append(rdma)
        for p in range(N_DEV):
            @pl.when(p == my_pos)
            def _():
                gather_ref[p] = comm_b[:, :]
        for p in range(N_DEV):
            @pl.when(p != my_pos)
            def _():
                pltpu.make_async_remote_copy(
                    src_ref=comm_b,
                    dst_ref=gather_ref.at[p],
                    send_sem=send_b.at[p],
                    recv_sem=recv_b.at[p],
                    device_id=(p,),
                    device_id_type=pl.DeviceIdType.MESH,
                ).wait_recv()
        out_ref[:, :] = gather_ref[:, :, :].reshape(
            ROWS, D_MODEL).astype(jnp.float32)
        for rdma in sends_a + sends_b:
            rdma.wait_send()

    out = pl.pallas_call(
        body,
        out_shape=jax.ShapeDtypeStruct((ROWS, D_MODEL), jnp.float32),
        in_specs=[pl.BlockSpec(memory_space=pltpu.VMEM)] * 5,
        out_specs=pl.BlockSpec(memory_space=pltpu.VMEM),
        scratch_shapes=[
            pltpu.VMEM((N_DEV, ROWS_PER, D_MODEL), jnp.bfloat16),
            pltpu.VMEM((N_DEV, ROWS_PER, D_MODEL), jnp.bfloat16),
            pltpu.VMEM((ROWS_PER, D_MODEL), jnp.bfloat16),
            pltpu.VMEM((N_DEV, ROWS_PER, D_MODEL), jnp.bfloat16),
            pltpu.SemaphoreType.DMA((1,)),
            pltpu.SemaphoreType.DMA((N_DEV,)),
            pltpu.SemaphoreType.DMA((N_DEV,)),
            pltpu.SemaphoreType.DMA((N_DEV,)),
            pltpu.SemaphoreType.DMA((N_DEV,)),
        ],
    )(xf, Wq, Ks, Vs, Wo)
    return out.reshape(B, SQ, D_MODEL)
